# Initial kernel scaffold; baseline (speedup 1.0000x reference)
import jax
import jax.numpy as jnp
from jax.experimental import pallas as pl


def kernel(x, edge_index, W1, b1, W2, b2, W3, b3):
    raise NotImplementedError("write your pallas kernel here")


# speedup vs baseline: 7.4210x; 7.4210x over previous
import functools

import jax
import jax.numpy as jnp
from jax import lax
from jax.experimental import pallas as pl
from jax.experimental.pallas import tpu as pltpu
from jax.experimental.pallas import tpu_sc as plsc

N_NODES = 10000
N_PAD = 10240
N_EDGES = 160000
N_TILES = 16
E_TILE = N_EDGES // N_TILES
EB = 2000
CAP = 10240

NB, NPASS, G, DW = 2560, 2, 128, 128
N_PP = NPASS
RPT = NB // N_TILES


def _compact_kernel():
  mesh = plsc.VectorSubcoreMesh(core_axis_name="c", subcore_axis_name="s")

  def body(src_hbm, dst_hbm, glist, llist, cnt,
           sbuf, dbuf, gcomp, lcomp, cbuf):
    c = lax.axis_index("c")
    s = lax.axis_index("s")
    iota = lax.iota(jnp.int32, 16)
    cvec = jnp.zeros((16,), jnp.int32)

    for pp in range(N_PP):
      base = (2 * pp + c) * NB

      def one_batch(b, cur):
        eoff = s * E_TILE + b * EB
        pltpu.sync_copy(src_hbm.at[pl.ds(eoff, EB)], sbuf)
        pltpu.sync_copy(dst_hbm.at[pl.ds(eoff, EB)], dbuf)

        def scan_group(g, cur):
          off = g * 16
          vs = sbuf[pl.ds(off, 16)]
          vd = dbuf[pl.ds(off, 16)]
          m = (vd >= base) & (vd < base + NB)
          pf = plsc.cumsum(jnp.where(m, 1, 0))
          pos = cur + pf - 1
          plsc.store_scatter(gcomp, [pos], vs, mask=m)
          plsc.store_scatter(lcomp, [pos], vd - base, mask=m)
          return cur + jnp.max(pf)

        return lax.fori_loop(0, EB // 16, scan_group, cur)

      cur = lax.fori_loop(0, E_TILE // EB, one_batch, jnp.int32(0))
      cvec = jnp.where(iota == pp, cur, cvec)
      pltpu.sync_copy(gcomp, glist.at[c, s, pp])
      pltpu.sync_copy(lcomp, llist.at[c, s, pp])

    cbuf[pl.ds(0, 16)] = cvec
    pltpu.sync_copy(cbuf, cnt.at[c, s])

  i32 = jnp.int32
  return pl.kernel(
      body,
      out_type=(
          jax.ShapeDtypeStruct((2, N_TILES, N_PP, CAP), i32),
          jax.ShapeDtypeStruct((2, N_TILES, N_PP, CAP), i32),
          jax.ShapeDtypeStruct((2, N_TILES, 16), i32),
      ),
      mesh=mesh,
      scratch_types=[
          pltpu.VMEM((EB,), i32),
          pltpu.VMEM((EB,), i32),
          pltpu.VMEM((CAP,), i32),
          pltpu.VMEM((CAP,), i32),
          pltpu.VMEM((16,), i32),
      ],
      compiler_params=pltpu.CompilerParams(needs_layout_passes=False),
  )


def _fire_kernel(do_gather, K):
  mesh = plsc.VectorSubcoreMesh(core_axis_name="c", subcore_axis_name="s")

  def body(*refs):
    if do_gather:
      (glist, llist, cnt, zrows, u_hbm, out_hbm,
       gbuf, lbuf, cbuf, gfire, dfire, stage, acc, sem) = refs
    else:
      (glist, llist, cnt, zrows, ones_hbm, out_hbm,
       gbuf, lbuf, cbuf, gfire, dfire, stage, acc, sem) = refs
    c = lax.axis_index("c")
    s = lax.axis_index("s")
    iota = lax.iota(jnp.int32, 16)

    pltpu.sync_copy(cnt.at[c, s], cbuf)
    if not do_gather:
      pltpu.sync_copy(ones_hbm, stage)
    cvec = cbuf[pl.ds(0, 16)]

    for p in range(NPASS):
      base = (2 * p + c) * NB
      if do_gather:
        pltpu.sync_copy(glist.at[c, s, p], gbuf)
      pltpu.sync_copy(llist.at[c, s, p], lbuf)
      cntv = cvec[p]
      nf = (cntv + (G - 1)) // G

      for k in range(K):
        pltpu.sync_copy(zrows, acc.at[pl.ds(s * RPT, RPT)])
        plsc.subcore_barrier()

        def fire(j, _, k=k):
          for off in range(0, G, 16):
            pos = j * G + off
            mv = (pos + iota) < cntv
            lv = lbuf[pl.ds(pos, 16)]
            dfire[pl.ds(off, 16)] = jnp.where(mv, lv, NB + s)
            if do_gather:
              gv = gbuf[pl.ds(pos, 16)]
              gfire[pl.ds(off, 16)] = jnp.where(mv, gv * K + k, -1)
          if do_gather:
            pltpu.async_copy(
                u_hbm.at[plsc.Indices(gfire, ignored_value=-1)], stage, sem
            ).wait()
          pltpu.sync_copy(stage, acc.at[dfire], add=True)
          return 0

        lax.fori_loop(0, nf, fire, 0)
        plsc.subcore_barrier()
        pltpu.sync_copy(
            acc.at[pl.ds(s * RPT, RPT)],
            out_hbm.at[pl.ds(base + s * RPT, RPT), pl.ds(k * DW, DW)],
        )
        plsc.subcore_barrier()

  i32, f32 = jnp.int32, jnp.float32
  return pl.kernel(
      body,
      out_type=jax.ShapeDtypeStruct((N_PAD, DW * K), f32),
      mesh=mesh,
      scratch_types=[
          pltpu.VMEM((CAP,), i32),
          pltpu.VMEM((CAP,), i32),
          pltpu.VMEM((16,), i32),
          pltpu.VMEM((G,), i32),
          pltpu.VMEM((G,), i32),
          pltpu.VMEM((G, DW), f32),
          pltpu.VMEM_SHARED((NB + 16, DW), f32),
          pltpu.SemaphoreType.DMA,
      ],
      compiler_params=pltpu.CompilerParams(needs_layout_passes=True),
  )


_BLK = 256
_GRID = N_PAD // _BLK


def _dinv(deg_blk):
  return lax.rsqrt(deg_blk[:, 0:1] + 1.0)


def _tc_scale_body(deg_ref, x_ref, u1_ref):
  u1_ref[...] = x_ref[...] * _dinv(deg_ref[...])


def _tc_layer1_body(deg_ref, t1_ref, u1_ref, w1_ref, b1_ref, u2_ref):
  dinv = _dinv(deg_ref[...])
  agg = (t1_ref[...] + u1_ref[...]) * dinv
  h = jnp.dot(agg, w1_ref[...], preferred_element_type=jnp.float32)
  h = jnp.maximum(h + b1_ref[...], 0.0)
  u2_ref[...] = h * dinv


def _tc_layer2_body(deg_ref, t2_ref, u2_ref, w2_ref, b2_ref, w3_ref, b3_ref,
                    o_ref):
  dinv = _dinv(deg_ref[...])
  agg = (t2_ref[...] + u2_ref[...]) * dinv
  h = jnp.dot(agg, w2_ref[...], preferred_element_type=jnp.float32)
  h = jnp.maximum(h + b2_ref[...], 0.0)
  o_ref[...] = jnp.dot(h, w3_ref[...], preferred_element_type=jnp.float32) \
      + b3_ref[...]


def _row_spec(d):
  return pl.BlockSpec((_BLK, d), lambda i: (i, 0))


def _full_spec(r, c):
  return pl.BlockSpec((r, c), lambda i: (0, 0))


def _tc_scale(deg, xp):
  return pl.pallas_call(
      _tc_scale_body,
      grid=(_GRID,),
      in_specs=[_row_spec(128), _row_spec(128)],
      out_specs=_row_spec(128),
      out_shape=jax.ShapeDtypeStruct((N_PAD, 128), jnp.float32),
  )(deg, xp)


def _tc_layer1(deg, t1, u1, w1p, b1):
  return pl.pallas_call(
      _tc_layer1_body,
      grid=(_GRID,),
      in_specs=[_row_spec(128), _row_spec(128), _row_spec(128),
                _full_spec(128, 1024), _full_spec(1, 1024)],
      out_specs=_row_spec(1024),
      out_shape=jax.ShapeDtypeStruct((N_PAD, 1024), jnp.float32),
  )(deg, t1, u1, w1p, b1)


def _tc_layer2(deg, t2, u2, w2, b2, w3p, b3p):
  return pl.pallas_call(
      _tc_layer2_body,
      grid=(_GRID,),
      in_specs=[_row_spec(128), _row_spec(1024), _row_spec(1024),
                _full_spec(1024, 1024), _full_spec(1, 1024),
                _full_spec(1024, 128), _full_spec(1, 128)],
      out_specs=_row_spec(128),
      out_shape=jax.ShapeDtypeStruct((N_PAD, 128), jnp.float32),
  )(deg, t2, u2, w2, b2, w3p, b3p)


@functools.lru_cache(maxsize=None)
def _sc_kernels():
  return (
      _compact_kernel(),
      _fire_kernel(do_gather=False, K=1),
      _fire_kernel(do_gather=True, K=1),
      _fire_kernel(do_gather=True, K=8),
  )


def kernel(x, edge_index, W1, b1, W2, b2, W3, b3):
  ei = edge_index.astype(jnp.int32)
  src, dst = ei[0], ei[1]

  f32 = jnp.float32
  xp = jnp.zeros((N_PAD, 128), f32).at[:N_NODES, :47].set(x)
  w1p = jnp.zeros((128, 1024), f32).at[:47, :].set(W1)
  w3p = jnp.zeros((1024, 128), f32).at[:, :47].set(W3)
  b3p = jnp.zeros((1, 128), f32).at[0, :47].set(b3)

  sc_compact, sc_deg, sc_agg128, sc_agg1k = _sc_kernels()
  zrows = jnp.zeros((RPT, DW), f32)
  ones_g = jnp.ones((G, DW), f32)

  glist, llist, cnt = sc_compact(src, dst)
  deg = sc_deg(glist, llist, cnt, zrows, ones_g)
  u1 = _tc_scale(deg, xp)
  t1 = sc_agg128(glist, llist, cnt, zrows, u1)
  u2 = _tc_layer1(deg, t1, u1, w1p, b1.reshape(1, 1024))
  t2 = sc_agg1k(glist, llist, cnt, zrows, u2.reshape(N_PAD * 8, 128))
  out = _tc_layer2(deg, t2, u2, W2, b2.reshape(1, 1024), w3p, b3p)
  return out[:N_NODES, :47]


# speedup vs baseline: 8.4822x; 1.1430x over previous
import functools

import jax
import jax.numpy as jnp
from jax import lax
from jax.experimental import pallas as pl
from jax.experimental.pallas import tpu as pltpu
from jax.experimental.pallas import tpu_sc as plsc

N_NODES = 10000
N_PAD = 10240
N_EDGES = 160000
N_TILES = 16
E_TILE = N_EDGES // N_TILES
EB = 2000
CAP = 10240

NB, NPASS, G, DW = 2560, 2, 128, 128
N_PP = NPASS
RPT = NB // N_TILES


def _compact_kernel():
  mesh = plsc.VectorSubcoreMesh(core_axis_name="c", subcore_axis_name="s")

  def body(src_hbm, dst_hbm, glist, llist, cnt,
           sbuf, dbuf, gcomp, lcomp, cbuf):
    c = lax.axis_index("c")
    s = lax.axis_index("s")
    iota = lax.iota(jnp.int32, 16)
    cvec = jnp.zeros((16,), jnp.int32)

    for pp in range(N_PP):
      base = (2 * pp + c) * NB

      def one_batch(b, cur):
        eoff = s * E_TILE + b * EB
        pltpu.sync_copy(src_hbm.at[pl.ds(eoff, EB)], sbuf)
        pltpu.sync_copy(dst_hbm.at[pl.ds(eoff, EB)], dbuf)

        def scan_group(g, cur):
          off = g * 16
          vs = sbuf[pl.ds(off, 16)]
          vd = dbuf[pl.ds(off, 16)]
          m = (vd >= base) & (vd < base + NB)
          pf = plsc.cumsum(jnp.where(m, 1, 0))
          pos = cur + pf - 1
          plsc.store_scatter(gcomp, [pos], vs, mask=m)
          plsc.store_scatter(lcomp, [pos], vd - base, mask=m)
          return cur + jnp.max(pf)

        return lax.fori_loop(0, EB // 16, scan_group, cur)

      cur = lax.fori_loop(0, E_TILE // EB, one_batch, jnp.int32(0))
      cvec = jnp.where(iota == pp, cur, cvec)
      pltpu.sync_copy(gcomp, glist.at[c, s, pp])
      pltpu.sync_copy(lcomp, llist.at[c, s, pp])

    cbuf[pl.ds(0, 16)] = cvec
    pltpu.sync_copy(cbuf, cnt.at[c, s])

  i32 = jnp.int32
  return pl.kernel(
      body,
      out_type=(
          jax.ShapeDtypeStruct((2, N_TILES, N_PP, CAP), i32),
          jax.ShapeDtypeStruct((2, N_TILES, N_PP, CAP), i32),
          jax.ShapeDtypeStruct((2, N_TILES, 16), i32),
      ),
      mesh=mesh,
      scratch_types=[
          pltpu.VMEM((EB,), i32),
          pltpu.VMEM((EB,), i32),
          pltpu.VMEM((CAP,), i32),
          pltpu.VMEM((CAP,), i32),
          pltpu.VMEM((16,), i32),
      ],
      compiler_params=pltpu.CompilerParams(needs_layout_passes=False),
  )


def _fire_kernel(do_gather, K):
  mesh = plsc.VectorSubcoreMesh(core_axis_name="c", subcore_axis_name="s")

  def body(*refs):
    if do_gather:
      (glist, llist, cnt, zrows, u_hbm, out_hbm, gbuf, lbuf, cbuf,
       gfire0, gfire1, dfire0, dfire1, stage0, stage1, acc,
       gsem0, gsem1, ssem0, ssem1) = refs
    else:
      (glist, llist, cnt, zrows, ones_hbm, out_hbm, gbuf, lbuf, cbuf,
       gfire0, gfire1, dfire0, dfire1, stage0, stage1, acc,
       gsem0, gsem1, ssem0, ssem1) = refs
    c = lax.axis_index("c")
    s = lax.axis_index("s")
    iota = lax.iota(jnp.int32, 16)

    pltpu.sync_copy(cnt.at[c, s], cbuf)
    if not do_gather:
      pltpu.sync_copy(ones_hbm, stage0)
      pltpu.sync_copy(ones_hbm, stage1)
    cvec = cbuf[pl.ds(0, 16)]

    def gather_start(gfire, stage, gsem):
      if do_gather:
        pltpu.async_copy(
            u_hbm.at[plsc.Indices(gfire, ignored_value=-1)], stage, gsem)

    def gather_wait(gfire, stage, gsem):
      if do_gather:
        pltpu.make_async_copy(
            u_hbm.at[plsc.Indices(gfire, ignored_value=-1)], stage, gsem
        ).wait()

    def scatter_wait(stage, dfire, ssem):
      pltpu.make_async_copy(stage, acc.at[dfire], ssem).wait()

    for p in range(NPASS):
      base = (2 * p + c) * NB
      if do_gather:
        pltpu.sync_copy(glist.at[c, s, p], gbuf)
      pltpu.sync_copy(llist.at[c, s, p], lbuf)
      cntv = cvec[p]
      nf2 = (cntv + (2 * G - 1)) // (2 * G)

      def chunk(k, _):
        pltpu.sync_copy(zrows, acc.at[pl.ds(s * RPT, RPT)])
        plsc.subcore_barrier()

        def build(j, gfire, dfire):
          for off in range(0, G, 16):
            pos = j * G + off
            mv = (pos + iota) < cntv
            lv = lbuf[pl.ds(pos, 16)]
            dfire[pl.ds(off, 16)] = jnp.where(mv, lv, NB + s)
            if do_gather:
              gv = gbuf[pl.ds(pos, 16)]
              gfire[pl.ds(off, 16)] = jnp.where(mv, gv * K + k, -1)

        def pair(t, _):
          pl.when(t > 0)(lambda: scatter_wait(stage0, dfire0, ssem0))
          build(2 * t, gfire0, dfire0)
          gather_start(gfire0, stage0, gsem0)
          pl.when(t > 0)(lambda: scatter_wait(stage1, dfire1, ssem1))
          build(2 * t + 1, gfire1, dfire1)
          gather_start(gfire1, stage1, gsem1)
          gather_wait(gfire0, stage0, gsem0)
          pltpu.async_copy(stage0, acc.at[dfire0], ssem0, add=True)
          gather_wait(gfire1, stage1, gsem1)
          pltpu.async_copy(stage1, acc.at[dfire1], ssem1, add=True)
          return 0

        lax.fori_loop(0, nf2, pair, 0)
        pl.when(nf2 > 0)(lambda: scatter_wait(stage0, dfire0, ssem0))
        pl.when(nf2 > 0)(lambda: scatter_wait(stage1, dfire1, ssem1))
        plsc.subcore_barrier()
        pltpu.sync_copy(
            acc.at[pl.ds(s * RPT, RPT)],
            out_hbm.at[pl.ds(base + s * RPT, RPT), pl.ds(k * DW, DW)],
        )
        plsc.subcore_barrier()
        return 0

      lax.fori_loop(0, K, chunk, 0)

  i32, f32 = jnp.int32, jnp.float32
  return pl.kernel(
      body,
      out_type=jax.ShapeDtypeStruct((N_PAD, DW * K), f32),
      mesh=mesh,
      scratch_types=[
          pltpu.VMEM((CAP,), i32),
          pltpu.VMEM((CAP,), i32),
          pltpu.VMEM((16,), i32),
          pltpu.VMEM((G,), i32),
          pltpu.VMEM((G,), i32),
          pltpu.VMEM((G,), i32),
          pltpu.VMEM((G,), i32),
          pltpu.VMEM((G, DW), f32),
          pltpu.VMEM((G, DW), f32),
          pltpu.VMEM_SHARED((NB + 16, DW), f32),
          pltpu.SemaphoreType.DMA,
          pltpu.SemaphoreType.DMA,
          pltpu.SemaphoreType.DMA,
          pltpu.SemaphoreType.DMA,
      ],
      compiler_params=pltpu.CompilerParams(needs_layout_passes=True),
  )


_BLK = 256
_GRID = N_PAD // _BLK


def _dinv(deg_blk):
  return lax.rsqrt(deg_blk[:, 0:1] + 1.0)


def _tc_scale_body(deg_ref, x_ref, u1_ref):
  u1_ref[...] = x_ref[...] * _dinv(deg_ref[...])


def _tc_layer1_body(deg_ref, t1_ref, u1_ref, w1_ref, b1_ref, u2_ref):
  dinv = _dinv(deg_ref[...])
  agg = (t1_ref[...] + u1_ref[...]) * dinv
  h = jnp.dot(agg, w1_ref[...], preferred_element_type=jnp.float32)
  h = jnp.maximum(h + b1_ref[...], 0.0)
  u2_ref[...] = h * dinv


def _tc_layer2_body(deg_ref, t2_ref, u2_ref, w2_ref, b2_ref, w3_ref, b3_ref,
                    o_ref):
  dinv = _dinv(deg_ref[...])
  agg = (t2_ref[...] + u2_ref[...]) * dinv
  h = jnp.dot(agg, w2_ref[...], preferred_element_type=jnp.float32)
  h = jnp.maximum(h + b2_ref[...], 0.0)
  o_ref[...] = jnp.dot(h, w3_ref[...], preferred_element_type=jnp.float32) \
      + b3_ref[...]


def _row_spec(d):
  return pl.BlockSpec((_BLK, d), lambda i: (i, 0))


def _full_spec(r, c):
  return pl.BlockSpec((r, c), lambda i: (0, 0))


def _tc_scale(deg, xp):
  return pl.pallas_call(
      _tc_scale_body,
      grid=(_GRID,),
      in_specs=[_row_spec(128), _row_spec(128)],
      out_specs=_row_spec(128),
      out_shape=jax.ShapeDtypeStruct((N_PAD, 128), jnp.float32),
  )(deg, xp)


def _tc_layer1(deg, t1, u1, w1p, b1):
  return pl.pallas_call(
      _tc_layer1_body,
      grid=(_GRID,),
      in_specs=[_row_spec(128), _row_spec(128), _row_spec(128),
                _full_spec(128, 1024), _full_spec(1, 1024)],
      out_specs=_row_spec(1024),
      out_shape=jax.ShapeDtypeStruct((N_PAD, 1024), jnp.float32),
  )(deg, t1, u1, w1p, b1)


def _tc_layer2(deg, t2, u2, w2, b2, w3p, b3p):
  return pl.pallas_call(
      _tc_layer2_body,
      grid=(_GRID,),
      in_specs=[_row_spec(128), _row_spec(1024), _row_spec(1024),
                _full_spec(1024, 1024), _full_spec(1, 1024),
                _full_spec(1024, 128), _full_spec(1, 128)],
      out_specs=_row_spec(128),
      out_shape=jax.ShapeDtypeStruct((N_PAD, 128), jnp.float32),
  )(deg, t2, u2, w2, b2, w3p, b3p)


@functools.lru_cache(maxsize=None)
def _sc_kernels():
  return (
      _compact_kernel(),
      _fire_kernel(do_gather=False, K=1),
      _fire_kernel(do_gather=True, K=1),
      _fire_kernel(do_gather=True, K=8),
  )


def kernel(x, edge_index, W1, b1, W2, b2, W3, b3):
  ei = edge_index.astype(jnp.int32)
  src, dst = ei[0], ei[1]

  f32 = jnp.float32
  xp = jnp.zeros((N_PAD, 128), f32).at[:N_NODES, :47].set(x)
  w1p = jnp.zeros((128, 1024), f32).at[:47, :].set(W1)
  w3p = jnp.zeros((1024, 128), f32).at[:, :47].set(W3)
  b3p = jnp.zeros((1, 128), f32).at[0, :47].set(b3)

  sc_compact, sc_deg, sc_agg128, sc_agg1k = _sc_kernels()
  zrows = jnp.zeros((RPT, DW), f32)
  ones_g = jnp.ones((G, DW), f32)

  glist, llist, cnt = sc_compact(src, dst)
  deg = sc_deg(glist, llist, cnt, zrows, ones_g)
  u1 = _tc_scale(deg, xp)
  t1 = sc_agg128(glist, llist, cnt, zrows, u1)
  u2 = _tc_layer1(deg, t1, u1, w1p, b1.reshape(1, 1024))
  t2 = sc_agg1k(glist, llist, cnt, zrows, u2.reshape(N_PAD * 8, 128))
  out = _tc_layer2(deg, t2, u2, W2, b2.reshape(1, 1024), w3p, b3p)
  return out[:N_NODES, :47]


# speedup vs baseline: 9.7523x; 1.1497x over previous
import functools

import jax
import jax.numpy as jnp
from jax import lax
from jax.experimental import pallas as pl
from jax.experimental.pallas import tpu as pltpu
from jax.experimental.pallas import tpu_sc as plsc

N_NODES = 10000
N_PAD = 10240
N_EDGES = 160000
N_TILES = 16
E_TILE = N_EDGES // N_TILES
EB = 2000
CAP = 10240

NB, NPASS, G, DW = 2560, 2, 128, 128
N_PP = NPASS
RPT = NB // N_TILES


def _compact_kernel():
  mesh = plsc.VectorSubcoreMesh(core_axis_name="c", subcore_axis_name="s")

  def body(src_hbm, dst_hbm, glist, llist, cnt,
           sbuf, dbuf, gcomp, lcomp, cbuf):
    c = lax.axis_index("c")
    s = lax.axis_index("s")
    iota = lax.iota(jnp.int32, 16)
    cvec = jnp.zeros((16,), jnp.int32)

    for pp in range(N_PP):
      base = (2 * pp + c) * NB

      def one_batch(b, cur):
        eoff = s * E_TILE + b * EB
        pltpu.sync_copy(src_hbm.at[pl.ds(eoff, EB)], sbuf)
        pltpu.sync_copy(dst_hbm.at[pl.ds(eoff, EB)], dbuf)

        def scan_group(g, cur):
          off = g * 16
          vs = sbuf[pl.ds(off, 16)]
          vd = dbuf[pl.ds(off, 16)]
          m = (vd >= base) & (vd < base + NB)
          pf = plsc.cumsum(jnp.where(m, 1, 0))
          pos = cur + pf - 1
          plsc.store_scatter(gcomp, [pos], vs, mask=m)
          plsc.store_scatter(lcomp, [pos], vd - base, mask=m)
          return cur + jnp.max(pf)

        return lax.fori_loop(0, EB // 16, scan_group, cur)

      cur = lax.fori_loop(0, E_TILE // EB, one_batch, jnp.int32(0))
      cvec = jnp.where(iota == pp, cur, cvec)
      pltpu.sync_copy(gcomp, glist.at[c, s, pp])
      pltpu.sync_copy(lcomp, llist.at[c, s, pp])

    cbuf[pl.ds(0, 16)] = cvec
    pltpu.sync_copy(cbuf, cnt.at[c, s])

  i32 = jnp.int32
  return pl.kernel(
      body,
      out_type=(
          jax.ShapeDtypeStruct((2, N_TILES, N_PP, CAP), i32),
          jax.ShapeDtypeStruct((2, N_TILES, N_PP, CAP), i32),
          jax.ShapeDtypeStruct((2, N_TILES, 16), i32),
      ),
      mesh=mesh,
      scratch_types=[
          pltpu.VMEM((EB,), i32),
          pltpu.VMEM((EB,), i32),
          pltpu.VMEM((CAP,), i32),
          pltpu.VMEM((CAP,), i32),
          pltpu.VMEM((16,), i32),
      ],
      compiler_params=pltpu.CompilerParams(needs_layout_passes=False),
  )


def _fire_kernel(do_gather, K):
  mesh = plsc.VectorSubcoreMesh(core_axis_name="c", subcore_axis_name="s")
  R = 4

  def body(*refs):
    (glist, llist, cnt, zrows, u_or_ones, out_hbm, gbuf, lbuf, cbuf) =         refs[:9]
    gfires = refs[9:9 + R]
    dfires = refs[9 + R:9 + 2 * R]
    stages = refs[9 + 2 * R:9 + 3 * R]
    acc = refs[9 + 3 * R]
    gsems = refs[10 + 3 * R:10 + 4 * R]
    ssems = refs[10 + 4 * R:10 + 5 * R]
    u_hbm = ones_hbm = u_or_ones
    c = lax.axis_index("c")
    s = lax.axis_index("s")
    iota = lax.iota(jnp.int32, 16)

    pltpu.sync_copy(cnt.at[c, s], cbuf)
    if not do_gather:
      for st in stages:
        pltpu.sync_copy(ones_hbm, st)
    cvec = cbuf[pl.ds(0, 16)]

    def gather_start(gfire, stage, gsem):
      if do_gather:
        pltpu.async_copy(
            u_hbm.at[plsc.Indices(gfire, ignored_value=-1)], stage, gsem)

    def gather_wait(gfire, stage, gsem):
      if do_gather:
        pltpu.make_async_copy(
            u_hbm.at[plsc.Indices(gfire, ignored_value=-1)], stage, gsem
        ).wait()

    def scatter_wait(stage, dfire, ssem):
      pltpu.make_async_copy(stage, acc.at[dfire], ssem).wait()

    for p in range(NPASS):
      base = (2 * p + c) * NB
      if do_gather:
        pltpu.sync_copy(glist.at[c, s, p], gbuf)
      pltpu.sync_copy(llist.at[c, s, p], lbuf)
      cntv = cvec[p]
      nrounds = (cntv + (R * G - 1)) // (R * G)

      def chunk(k, _):
        pltpu.sync_copy(zrows, acc.at[pl.ds(s * RPT, RPT)])
        plsc.subcore_barrier()

        def build(j, gfire, dfire):
          for off in range(0, G, 16):
            pos = j * G + off
            mv = (pos + iota) < cntv
            lv = lbuf[pl.ds(pos, 16)]
            dfire[pl.ds(off, 16)] = jnp.where(mv, lv, NB + s)
            if do_gather:
              gv = gbuf[pl.ds(pos, 16)]
              gfire[pl.ds(off, 16)] = jnp.where(mv, gv * K + k, -1)

        def ring(t, _):
          for r in range(R):
            pl.when(t > 0)(
                lambda r=r: scatter_wait(stages[r], dfires[r], ssems[r]))
            build(R * t + r, gfires[r], dfires[r])
            gather_start(gfires[r], stages[r], gsems[r])
          for r in range(R):
            gather_wait(gfires[r], stages[r], gsems[r])
            pltpu.async_copy(stages[r], acc.at[dfires[r]], ssems[r],
                             add=True)
          return 0

        lax.fori_loop(0, nrounds, ring, 0)
        for r in range(R):
          pl.when(nrounds > 0)(
              lambda r=r: scatter_wait(stages[r], dfires[r], ssems[r]))
        plsc.subcore_barrier()
        pltpu.sync_copy(
            acc.at[pl.ds(s * RPT, RPT)],
            out_hbm.at[pl.ds(base + s * RPT, RPT), pl.ds(k * DW, DW)],
        )
        plsc.subcore_barrier()
        return 0

      lax.fori_loop(0, K, chunk, 0)

  i32, f32 = jnp.int32, jnp.float32
  return pl.kernel(
      body,
      out_type=jax.ShapeDtypeStruct((N_PAD, DW * K), f32),
      mesh=mesh,
      scratch_types=(
          [pltpu.VMEM((CAP,), i32),
           pltpu.VMEM((CAP,), i32),
           pltpu.VMEM((16,), i32)]
          + [pltpu.VMEM((G,), i32) for _ in range(2 * R)]
          + [pltpu.VMEM((G, DW), f32) for _ in range(R)]
          + [pltpu.VMEM_SHARED((NB + 16, DW), f32)]
          + [pltpu.SemaphoreType.DMA for _ in range(2 * R)]
      ),
      compiler_params=pltpu.CompilerParams(needs_layout_passes=True),
  )


_BLK = 256
_GRID = N_PAD // _BLK


def _dinv(deg_blk):
  return lax.rsqrt(deg_blk[:, 0:1] + 1.0)


def _tc_scale_body(deg_ref, x_ref, u1_ref):
  u1_ref[...] = x_ref[...] * _dinv(deg_ref[...])


def _tc_layer1_body(deg_ref, t1_ref, u1_ref, w1_ref, b1_ref, u2_ref):
  dinv = _dinv(deg_ref[...])
  agg = (t1_ref[...] + u1_ref[...]) * dinv
  h = jnp.dot(agg.astype(jnp.bfloat16), w1_ref[...],
              preferred_element_type=jnp.float32)
  h = jnp.maximum(h + b1_ref[...], 0.0)
  u2_ref[...] = h * dinv


def _tc_layer2_body(deg_ref, t2_ref, u2_ref, w2_ref, b2_ref, w3_ref, b3_ref,
                    o_ref):
  dinv = _dinv(deg_ref[...])
  agg = (t2_ref[...] + u2_ref[...]) * dinv
  h = jnp.dot(agg.astype(jnp.bfloat16), w2_ref[...],
              preferred_element_type=jnp.float32)
  h = jnp.maximum(h + b2_ref[...], 0.0)
  o_ref[...] = jnp.dot(h.astype(jnp.bfloat16), w3_ref[...],
                       preferred_element_type=jnp.float32) + b3_ref[...]


def _row_spec(d):
  return pl.BlockSpec((_BLK, d), lambda i: (i, 0))


def _full_spec(r, c):
  return pl.BlockSpec((r, c), lambda i: (0, 0))


def _tc_scale(deg, xp):
  return pl.pallas_call(
      _tc_scale_body,
      grid=(_GRID,),
      in_specs=[_row_spec(128), _row_spec(128)],
      out_specs=_row_spec(128),
      out_shape=jax.ShapeDtypeStruct((N_PAD, 128), jnp.float32),
  )(deg, xp)


def _tc_layer1(deg, t1, u1, w1p, b1):
  return pl.pallas_call(
      _tc_layer1_body,
      grid=(_GRID,),
      in_specs=[_row_spec(128), _row_spec(128), _row_spec(128),
                _full_spec(128, 1024), _full_spec(1, 1024)],
      out_specs=_row_spec(1024),
      out_shape=jax.ShapeDtypeStruct((N_PAD, 1024), jnp.float32),
  )(deg, t1, u1, w1p, b1)


def _tc_layer2(deg, t2, u2, w2, b2, w3p, b3p):
  return pl.pallas_call(
      _tc_layer2_body,
      grid=(_GRID,),
      in_specs=[_row_spec(128), _row_spec(1024), _row_spec(1024),
                _full_spec(1024, 1024), _full_spec(1, 1024),
                _full_spec(1024, 128), _full_spec(1, 128)],
      out_specs=_row_spec(128),
      out_shape=jax.ShapeDtypeStruct((N_PAD, 128), jnp.float32),
  )(deg, t2, u2, w2, b2, w3p, b3p)


@functools.lru_cache(maxsize=None)
def _sc_kernels():
  return (
      _compact_kernel(),
      _fire_kernel(do_gather=False, K=1),
      _fire_kernel(do_gather=True, K=1),
      _fire_kernel(do_gather=True, K=8),
  )


def kernel(x, edge_index, W1, b1, W2, b2, W3, b3):
  ei = edge_index.astype(jnp.int32)
  src, dst = ei[0], ei[1]

  f32 = jnp.float32
  xp = jnp.zeros((N_PAD, 128), f32).at[:N_NODES, :47].set(x)
  w1p = jnp.zeros((128, 1024), f32).at[:47, :].set(W1)
  w3p = jnp.zeros((1024, 128), f32).at[:, :47].set(W3)
  b3p = jnp.zeros((1, 128), f32).at[0, :47].set(b3)

  sc_compact, sc_deg, sc_agg128, sc_agg1k = _sc_kernels()
  zrows = jnp.zeros((RPT, DW), f32)
  ones_g = jnp.ones((G, DW), f32)

  glist, llist, cnt = sc_compact(src, dst)
  deg = sc_deg(glist, llist, cnt, zrows, ones_g)
  u1 = _tc_scale(deg, xp)
  t1 = sc_agg128(glist, llist, cnt, zrows, u1)
  u2 = _tc_layer1(deg, t1, u1, w1p.astype(jnp.bfloat16),
                  b1.reshape(1, 1024))
  t2 = sc_agg1k(glist, llist, cnt, zrows, u2.reshape(N_PAD * 8, 128))
  out = _tc_layer2(deg, t2, u2, W2.astype(jnp.bfloat16),
                   b2.reshape(1, 1024), w3p.astype(jnp.bfloat16), b3p)
  return out[:N_NODES, :47]


# speedup vs baseline: 9.7533x; 1.0001x over previous
import functools

import jax
import jax.numpy as jnp
from jax import lax
from jax.experimental import pallas as pl
from jax.experimental.pallas import tpu as pltpu
from jax.experimental.pallas import tpu_sc as plsc

N_NODES = 10000
N_PAD = 10240
N_EDGES = 160000
N_TILES = 16
E_TILE = N_EDGES // N_TILES
EB = 2000
CAP = 10240

NB, NPASS, G, DW = 2560, 2, 128, 128
N_PP = NPASS
RPT = NB // N_TILES


def _compact_kernel():
  mesh = plsc.VectorSubcoreMesh(core_axis_name="c", subcore_axis_name="s")

  def body(src_hbm, dst_hbm, glist, llist, cnt,
           sbuf, dbuf, gcomp, lcomp, cbuf):
    c = lax.axis_index("c")
    s = lax.axis_index("s")
    iota = lax.iota(jnp.int32, 16)
    cvec = jnp.zeros((16,), jnp.int32)

    for pp in range(N_PP):
      base = (2 * pp + c) * NB

      def one_batch(b, cur):
        eoff = s * E_TILE + b * EB
        pltpu.sync_copy(src_hbm.at[pl.ds(eoff, EB)], sbuf)
        pltpu.sync_copy(dst_hbm.at[pl.ds(eoff, EB)], dbuf)

        def scan_group(g, cur):
          off = g * 16
          vs = sbuf[pl.ds(off, 16)]
          vd = dbuf[pl.ds(off, 16)]
          m = (vd >= base) & (vd < base + NB)
          pf = plsc.cumsum(jnp.where(m, 1, 0))
          pos = cur + pf - 1
          plsc.store_scatter(gcomp, [pos], vs, mask=m)
          plsc.store_scatter(lcomp, [pos], vd - base, mask=m)
          return cur + jnp.max(pf)

        return lax.fori_loop(0, EB // 16, scan_group, cur)

      cur = lax.fori_loop(0, E_TILE // EB, one_batch, jnp.int32(0))
      cvec = jnp.where(iota == pp, cur, cvec)
      pltpu.sync_copy(gcomp, glist.at[c, s, pp])
      pltpu.sync_copy(lcomp, llist.at[c, s, pp])

    cbuf[pl.ds(0, 16)] = cvec
    pltpu.sync_copy(cbuf, cnt.at[c, s])

  i32 = jnp.int32
  return pl.kernel(
      body,
      out_type=(
          jax.ShapeDtypeStruct((2, N_TILES, N_PP, CAP), i32),
          jax.ShapeDtypeStruct((2, N_TILES, N_PP, CAP), i32),
          jax.ShapeDtypeStruct((2, N_TILES, 16), i32),
      ),
      mesh=mesh,
      scratch_types=[
          pltpu.VMEM((EB,), i32),
          pltpu.VMEM((EB,), i32),
          pltpu.VMEM((CAP,), i32),
          pltpu.VMEM((CAP,), i32),
          pltpu.VMEM((16,), i32),
      ],
      compiler_params=pltpu.CompilerParams(needs_layout_passes=False),
  )


def _fire_kernel(do_gather, K, R=4):
  mesh = plsc.VectorSubcoreMesh(core_axis_name="c", subcore_axis_name="s")

  def body(*refs):
    (glist, llist, cnt, zrows, u_or_ones, out_hbm, gbuf, lbuf, cbuf) =         refs[:9]
    gfires = refs[9:9 + R]
    dfires = refs[9 + R:9 + 2 * R]
    stages = refs[9 + 2 * R:9 + 3 * R]
    acc = refs[9 + 3 * R]
    gsems = refs[10 + 3 * R:10 + 4 * R]
    ssems = refs[10 + 4 * R:10 + 5 * R]
    u_hbm = ones_hbm = u_or_ones
    c = lax.axis_index("c")
    s = lax.axis_index("s")
    iota = lax.iota(jnp.int32, 16)

    pltpu.sync_copy(cnt.at[c, s], cbuf)
    if not do_gather:
      for st in stages:
        pltpu.sync_copy(ones_hbm, st)
    cvec = cbuf[pl.ds(0, 16)]

    def gather_start(gfire, stage, gsem):
      if do_gather:
        pltpu.async_copy(
            u_hbm.at[plsc.Indices(gfire, ignored_value=-1)], stage, gsem)

    def gather_wait(gfire, stage, gsem):
      if do_gather:
        pltpu.make_async_copy(
            u_hbm.at[plsc.Indices(gfire, ignored_value=-1)], stage, gsem
        ).wait()

    def scatter_wait(stage, dfire, ssem):
      pltpu.make_async_copy(stage, acc.at[dfire], ssem).wait()

    for p in range(NPASS):
      base = (2 * p + c) * NB
      if do_gather:
        pltpu.sync_copy(glist.at[c, s, p], gbuf)
      pltpu.sync_copy(llist.at[c, s, p], lbuf)
      cntv = cvec[p]
      nrounds = (cntv + (R * G - 1)) // (R * G)

      def chunk(k, _):
        pltpu.sync_copy(zrows, acc.at[pl.ds(s * RPT, RPT)])
        plsc.subcore_barrier()

        def build(j, gfire, dfire):
          for off in range(0, G, 16):
            pos = j * G + off
            mv = (pos + iota) < cntv
            lv = lbuf[pl.ds(pos, 16)]
            dfire[pl.ds(off, 16)] = jnp.where(mv, lv, NB + s)
            if do_gather:
              gv = gbuf[pl.ds(pos, 16)]
              gfire[pl.ds(off, 16)] = jnp.where(mv, gv * K + k, -1)

        def ring(t, _):
          for r in range(R):
            pl.when(t > 0)(
                lambda r=r: scatter_wait(stages[r], dfires[r], ssems[r]))
            build(R * t + r, gfires[r], dfires[r])
            gather_start(gfires[r], stages[r], gsems[r])
          for r in range(R):
            gather_wait(gfires[r], stages[r], gsems[r])
            pltpu.async_copy(stages[r], acc.at[dfires[r]], ssems[r],
                             add=True)
          return 0

        lax.fori_loop(0, nrounds, ring, 0)
        for r in range(R):
          pl.when(nrounds > 0)(
              lambda r=r: scatter_wait(stages[r], dfires[r], ssems[r]))
        plsc.subcore_barrier()
        pltpu.sync_copy(
            acc.at[pl.ds(s * RPT, RPT)],
            out_hbm.at[pl.ds(base + s * RPT, RPT), pl.ds(k * DW, DW)],
        )
        plsc.subcore_barrier()
        return 0

      lax.fori_loop(0, K, chunk, 0)

  i32, f32 = jnp.int32, jnp.float32
  return pl.kernel(
      body,
      out_type=jax.ShapeDtypeStruct((N_PAD, DW * K), f32),
      mesh=mesh,
      scratch_types=(
          [pltpu.VMEM((CAP,), i32),
           pltpu.VMEM((CAP,), i32),
           pltpu.VMEM((16,), i32)]
          + [pltpu.VMEM((G,), i32) for _ in range(2 * R)]
          + [pltpu.VMEM((G, DW), f32) for _ in range(R)]
          + [pltpu.VMEM_SHARED((NB + 16, DW), f32)]
          + [pltpu.SemaphoreType.DMA for _ in range(2 * R)]
      ),
      compiler_params=pltpu.CompilerParams(needs_layout_passes=True),
  )


_BLK = 256
_GRID = N_PAD // _BLK


def _dinv(deg_blk):
  return lax.rsqrt(deg_blk[:, 0:1] + 1.0)


def _tc_scale_body(deg_ref, x_ref, u1_ref):
  u1_ref[...] = x_ref[...] * _dinv(deg_ref[...])


def _tc_layer1_body(deg_ref, t1_ref, u1_ref, w1_ref, b1_ref, u2_ref):
  dinv = _dinv(deg_ref[...])
  agg = (t1_ref[...] + u1_ref[...]) * dinv
  h = jnp.dot(agg.astype(jnp.bfloat16), w1_ref[...],
              preferred_element_type=jnp.float32)
  h = jnp.maximum(h + b1_ref[...], 0.0)
  u2_ref[...] = h * dinv


def _tc_layer2_body(deg_ref, t2_ref, u2_ref, w2_ref, b2_ref, w3_ref, b3_ref,
                    o_ref):
  dinv = _dinv(deg_ref[...])
  agg = (t2_ref[...] + u2_ref[...]) * dinv
  h = jnp.dot(agg.astype(jnp.bfloat16), w2_ref[...],
              preferred_element_type=jnp.float32)
  h = jnp.maximum(h + b2_ref[...], 0.0)
  o_ref[...] = jnp.dot(h.astype(jnp.bfloat16), w3_ref[...],
                       preferred_element_type=jnp.float32) + b3_ref[...]


def _row_spec(d):
  return pl.BlockSpec((_BLK, d), lambda i: (i, 0))


def _full_spec(r, c):
  return pl.BlockSpec((r, c), lambda i: (0, 0))


def _tc_scale(deg, xp):
  return pl.pallas_call(
      _tc_scale_body,
      grid=(_GRID,),
      in_specs=[_row_spec(128), _row_spec(128)],
      out_specs=_row_spec(128),
      out_shape=jax.ShapeDtypeStruct((N_PAD, 128), jnp.float32),
  )(deg, xp)


def _tc_layer1(deg, t1, u1, w1p, b1):
  return pl.pallas_call(
      _tc_layer1_body,
      grid=(_GRID,),
      in_specs=[_row_spec(128), _row_spec(128), _row_spec(128),
                _full_spec(128, 1024), _full_spec(1, 1024)],
      out_specs=_row_spec(1024),
      out_shape=jax.ShapeDtypeStruct((N_PAD, 1024), jnp.float32),
  )(deg, t1, u1, w1p, b1)


def _tc_layer2(deg, t2, u2, w2, b2, w3p, b3p):
  return pl.pallas_call(
      _tc_layer2_body,
      grid=(_GRID,),
      in_specs=[_row_spec(128), _row_spec(1024), _row_spec(1024),
                _full_spec(1024, 1024), _full_spec(1, 1024),
                _full_spec(1024, 128), _full_spec(1, 128)],
      out_specs=_row_spec(128),
      out_shape=jax.ShapeDtypeStruct((N_PAD, 128), jnp.float32),
  )(deg, t2, u2, w2, b2, w3p, b3p)


@functools.lru_cache(maxsize=None)
def _sc_kernels():
  return (
      _compact_kernel(),
      _fire_kernel(do_gather=False, K=1),
      _fire_kernel(do_gather=True, K=1),
      _fire_kernel(do_gather=True, K=8),
  )


def kernel(x, edge_index, W1, b1, W2, b2, W3, b3):
  ei = edge_index.astype(jnp.int32)
  src, dst = ei[0], ei[1]

  f32 = jnp.float32
  xp = jnp.zeros((N_PAD, 128), f32).at[:N_NODES, :47].set(x)
  w1p = jnp.zeros((128, 1024), f32).at[:47, :].set(W1)
  w3p = jnp.zeros((1024, 128), f32).at[:, :47].set(W3)
  b3p = jnp.zeros((1, 128), f32).at[0, :47].set(b3)

  sc_compact, sc_deg, sc_agg128, sc_agg1k = _sc_kernels()
  zrows = jnp.zeros((RPT, DW), f32)
  ones_g = jnp.ones((G, DW), f32)

  glist, llist, cnt = sc_compact(src, dst)
  deg = sc_deg(glist, llist, cnt, zrows, ones_g)
  u1 = _tc_scale(deg, xp)
  t1 = sc_agg128(glist, llist, cnt, zrows, u1)
  u2 = _tc_layer1(deg, t1, u1, w1p.astype(jnp.bfloat16),
                  b1.reshape(1, 1024))
  t2 = sc_agg1k(glist, llist, cnt, zrows, u2.reshape(N_PAD * 8, 128))
  out = _tc_layer2(deg, t2, u2, W2.astype(jnp.bfloat16),
                   b2.reshape(1, 1024), w3p.astype(jnp.bfloat16), b3p)
  return out[:N_NODES, :47]
